# baseline (speedup 1.0000x reference)
import jax
import jax.numpy as jnp
from jax.experimental import pallas as pl
from jax.experimental.pallas import tpu as pltpu

_MAX_SIZE = 50
_HALF = 128
_BATCH = 64
_NSEM = 8


def _pos_broadcast_kernel(row_ref, col_ref, out_ref, scratch):
    @pl.when(pl.program_id(0) == 0)
    def _build():
        col = col_ref[...]
        for i in range(_MAX_SIZE):
            scratch[pl.ds(i * _MAX_SIZE, _MAX_SIZE), :_HALF] = (
                jnp.broadcast_to(row_ref[pl.ds(i, 1), :], (_MAX_SIZE, _HALF))
            )
            scratch[pl.ds(i * _MAX_SIZE, _MAX_SIZE), _HALF:] = col

    out_ref[0] = scratch[...]


def kernel(batch_size, row_embed, col_embed):
    zero = (jnp.asarray(batch_size) - _BATCH).astype(row_embed.dtype)
    row_embed = row_embed + zero
    col_embed = col_embed + zero

    return pl.pallas_call(
        _pos_broadcast_kernel,
        grid=(_BATCH,),
        in_specs=[
            pl.BlockSpec((_MAX_SIZE, _HALF), lambda b: (0, 0)),
            pl.BlockSpec((_MAX_SIZE, _HALF), lambda b: (0, 0)),
        ],
        out_specs=pl.BlockSpec(
            (1, _MAX_SIZE * _MAX_SIZE, 2 * _HALF), lambda b: (b, 0, 0)
        ),
        out_shape=jax.ShapeDtypeStruct(
            (_BATCH, _MAX_SIZE * _MAX_SIZE, 2 * _HALF), row_embed.dtype
        ),
        scratch_shapes=[
            pltpu.VMEM((_MAX_SIZE * _MAX_SIZE, 2 * _HALF), row_embed.dtype),
        ],
    )(row_embed, col_embed)


# speedup vs baseline: 1.0203x; 1.0203x over previous
import jax
import jax.numpy as jnp
from jax.experimental import pallas as pl
from jax.experimental.pallas import tpu as pltpu

_MAX_SIZE = 50
_HALF = 128
_BATCH = 64
_NSEM = 8


_NSRC = 4


def _pos_broadcast_kernel(row_ref, col_ref, out_ref, scratch, sems):
    col = col_ref[...]
    for k in range(_NSRC):
        for i in range(_MAX_SIZE):
            scratch[k, pl.ds(i * _MAX_SIZE, _MAX_SIZE), :_HALF] = (
                jnp.broadcast_to(row_ref[pl.ds(i, 1), :], (_MAX_SIZE, _HALF))
            )
            scratch[k, pl.ds(i * _MAX_SIZE, _MAX_SIZE), _HALF:] = col

    for b in range(_BATCH):
        pltpu.make_async_copy(
            scratch.at[b % _NSRC], out_ref.at[b], sems.at[b % _NSEM]
        ).start()
    for b in range(_BATCH):
        pltpu.make_async_copy(
            scratch.at[b % _NSRC], out_ref.at[b], sems.at[b % _NSEM]
        ).wait()


def kernel(batch_size, row_embed, col_embed):
    zero = (jnp.asarray(batch_size) - _BATCH).astype(row_embed.dtype)
    row_embed = row_embed + zero
    col_embed = col_embed + zero

    return pl.pallas_call(
        _pos_broadcast_kernel,
        in_specs=[
            pl.BlockSpec(memory_space=pltpu.MemorySpace.VMEM),
            pl.BlockSpec(memory_space=pltpu.MemorySpace.VMEM),
        ],
        out_specs=pl.BlockSpec(memory_space=pltpu.MemorySpace.HBM),
        out_shape=jax.ShapeDtypeStruct(
            (_BATCH, _MAX_SIZE * _MAX_SIZE, 2 * _HALF), row_embed.dtype
        ),
        scratch_shapes=[
            pltpu.VMEM(
                (_NSRC, _MAX_SIZE * _MAX_SIZE, 2 * _HALF), row_embed.dtype
            ),
            pltpu.SemaphoreType.DMA((_NSEM,)),
        ],
    )(row_embed, col_embed)


# speedup vs baseline: 4.6994x; 4.6060x over previous
import jax
import jax.numpy as jnp
from jax.experimental import pallas as pl
from jax.experimental.pallas import tpu as pltpu

_MAX_SIZE = 50
_HALF = 128
_BATCH = 64


def _pos_broadcast_kernel(row_ref, col_ref, out_ref):
    i = pl.program_id(0)
    row = row_ref[pl.ds(i, 1), :]
    col = col_ref[...]
    out_ref[:, :, :_HALF] = jnp.broadcast_to(
        row[:, None, :], (_MAX_SIZE, _BATCH, _HALF)
    )
    out_ref[:, :, _HALF:] = jnp.broadcast_to(
        col[:, None, :], (_MAX_SIZE, _BATCH, _HALF)
    )


def kernel(batch_size, row_embed, col_embed):
    zero = (jnp.asarray(batch_size) - _BATCH).astype(row_embed.dtype)
    row_embed = row_embed + zero
    col_embed = col_embed + zero

    out = pl.pallas_call(
        _pos_broadcast_kernel,
        grid=(_MAX_SIZE,),
        in_specs=[
            pl.BlockSpec((_MAX_SIZE, _HALF), lambda i: (0, 0)),
            pl.BlockSpec((_MAX_SIZE, _HALF), lambda i: (0, 0)),
        ],
        out_specs=pl.BlockSpec(
            (_MAX_SIZE, _BATCH, 2 * _HALF), lambda i: (i, 0, 0)
        ),
        out_shape=jax.ShapeDtypeStruct(
            (_MAX_SIZE * _MAX_SIZE, _BATCH, 2 * _HALF), row_embed.dtype
        ),
    )(row_embed, col_embed)
    return jnp.transpose(out, (1, 0, 2))
